# baseline (speedup 1.0000x reference)
import functools

import jax
import jax.numpy as jnp
from jax import lax
from jax.experimental import pallas as pl
from jax.experimental.pallas import tpu as pltpu
from jax.experimental.pallas import tpu_sc as plsc

NCORE = 2
NSUB = 16
LANES = 16


def _edge_mlp_body(rbf_ref, ev_ref, w1t_ref, b1_ref, w2t_ref, b2_ref,
                   fs_ref, fv1_ref, evf0_ref, evf1_ref, evf2_ref, *, H):
    rbf = rbf_ref[...]
    h = jnp.dot(rbf, w1t_ref[...], preferred_element_type=jnp.float32) + b1_ref[...]
    h = h * jax.nn.sigmoid(h)
    fw = jnp.dot(h, w2t_ref[...], preferred_element_type=jnp.float32) + b2_ref[...]
    fs_ref[...] = fw[:, :H]
    fv1_ref[...] = fw[:, H:2 * H]
    fv2 = fw[:, 2 * H:]
    ev = ev_ref[...]
    evf0_ref[...] = ev[:, 0:1] * fv2
    evf1_ref[...] = ev[:, 1:2] * fv2
    evf2_ref[...] = ev[:, 2:3] * fv2


def _edge_mlp(rbf_p, ev_p, W1, b1, W2, b2, *, E_pad, H, R):
    BE = 512
    grid = (E_pad // BE,)
    full = lambda shape: pl.BlockSpec(shape, lambda i: (0, 0))
    out_sd = jax.ShapeDtypeStruct((E_pad, H), jnp.float32)
    return pl.pallas_call(
        functools.partial(_edge_mlp_body, H=H),
        grid=grid,
        in_specs=[
            pl.BlockSpec((BE, R), lambda i: (i, 0)),
            pl.BlockSpec((BE, 3), lambda i: (i, 0)),
            full((R, H)),
            full((1, H)),
            full((H, 3 * H)),
            full((1, 3 * H)),
        ],
        out_specs=[pl.BlockSpec((BE, H), lambda i: (i, 0))] * 5,
        out_shape=[out_sd] * 5,
    )(rbf_p, ev_p, W1.T, b1[None, :], W2.T, b2[None, :])


def _make_sc_scatter(E_pad, N_pad, H, B):
    EPT = E_pad // NSUB
    NBLK = EPT // B
    NGRP = NBLK // 4
    GBLK = E_pad // B
    RPT = N_pad // NSUB
    mesh = plsc.VectorSubcoreMesh(core_axis_name="c", subcore_axis_name="s",
                                  num_cores=NCORE, num_subcores=NSUB)

    @functools.partial(
        pl.kernel,
        out_type=jax.ShapeDtypeStruct((4, N_pad, H), jnp.float32),
        mesh=mesh,
        scratch_types=[
            pltpu.VMEM((4, 2, B), jnp.int32),
            pltpu.VMEM((2, B, H), jnp.float32),
            pltpu.VMEM((2, B, H), jnp.float32),
            pltpu.VMEM((2, B, H), jnp.float32),
            pltpu.VMEM_SHARED((N_pad, H), jnp.float32),
            pltpu.SemaphoreType.DMA, pltpu.SemaphoreType.DMA,
            pltpu.SemaphoreType.DMA, pltpu.SemaphoreType.DMA,
            pltpu.SemaphoreType.DMA, pltpu.SemaphoreType.DMA,
            pltpu.SemaphoreType.DMA, pltpu.SemaphoreType.DMA,
            pltpu.SemaphoreType.DMA, pltpu.SemaphoreType.DMA,
        ],
    )
    def sc_scatter(idx_hbm, zeros_hbm,
                   sf_hbm, vf0_hbm, vf1_hbm, vf2_hbm,
                   fs_hbm, fv1_hbm, evf0_hbm, evf1_hbm, evf2_hbm,
                   out_hbm,
                   idx_v, rows_v, mul_v, add_v, acc_sh,
                   sg0, sg1, sm0, sm1, sa0, sa1, ss0, ss1, si0, si1):
        c = lax.axis_index("c")
        s = lax.axis_index("s")
        sg = (sg0, sg1)
        sm = (sm0, sm1)
        sa = (sa0, sa1)
        ss = (ss0, ss1)
        si = (si0, si1)

        def run_pass(p, feat_hbm, mul_hbm, add_hbm):
            tb = s * EPT
            tg = s * NBLK
            pltpu.sync_copy(zeros_hbm.at[pl.ds(s * RPT, RPT)],
                            acc_sh.at[pl.ds(s * RPT, RPT)])
            plsc.subcore_barrier()

            def issue_idx(i, b4):
                pltpu.async_copy(idx_hbm.at[tg + i], idx_v.at[b4], si[b4 % 2])

            def wait_idx(b4):
                pltpu.make_async_copy(idx_hbm.at[0], idx_v.at[b4],
                                      si[b4 % 2]).wait()

            def issue_in(i, b4, bb):
                base = tb + i * B
                pltpu.async_copy(feat_hbm.at[idx_v.at[b4, 0]], rows_v.at[bb],
                                 sg[bb])
                pltpu.async_copy(mul_hbm.at[pl.ds(base, B)], mul_v.at[bb],
                                 sm[bb])
                if add_hbm is not None:
                    pltpu.async_copy(add_hbm.at[pl.ds(base, B)], add_v.at[bb],
                                     sa[bb])

            def wait_in(bb):
                pltpu.make_async_copy(feat_hbm.at[pl.ds(0, B)], rows_v.at[bb],
                                      sg[bb]).wait()
                pltpu.make_async_copy(mul_hbm.at[pl.ds(0, B)], mul_v.at[bb],
                                      sm[bb]).wait()
                if add_hbm is not None:
                    pltpu.make_async_copy(add_hbm.at[pl.ds(0, B)],
                                          add_v.at[bb], sa[bb]).wait()

            def wait_scatter(bb):
                pltpu.make_async_copy(zeros_hbm.at[pl.ds(0, B)],
                                      rows_v.at[bb], ss[bb]).wait()

            def compute(bb):
                @plsc.parallel_loop(0, B, step=1, unroll=2)
                def row_fn(r):
                    for j in range(H // LANES):
                        sl = pl.ds(j * LANES, LANES)
                        m = rows_v[bb, r, sl] * mul_v[bb, r, sl]
                        if add_hbm is not None:
                            m = m + add_v[bb, r, sl]
                        rows_v[bb, r, sl] = m

            def step(i, b4, first):
                bb = b4 % 2
                nb = (b4 + 1) % 2
                nb4 = (b4 + 1) % 4
                nn4 = (b4 + 2) % 4
                i_next = lax.rem(i + 1, NBLK)
                i_next2 = lax.rem(i + 2, NBLK)
                issue_idx(i_next2, nn4)
                if not first:
                    wait_scatter(nb)
                wait_idx(nb4)
                issue_in(i_next, nb4, nb)
                wait_in(bb)
                compute(bb)
                pltpu.async_copy(rows_v.at[bb], acc_sh.at[idx_v.at[b4, 1]],
                                 ss[bb], add=True)

            issue_idx(0, 0)
            wait_idx(0)
            issue_in(0, 0, 0)
            issue_idx(1, 1)
            for b in range(4):
                step(jnp.int32(b), b, first=(b == 0))

            def grp(g, carry):
                for b in range(4):
                    step(g * 4 + b, b, first=False)
                return carry
            lax.fori_loop(1, NGRP, grp, 0)

            wait_in(0)
            wait_idx(1)
            wait_scatter(1)

            plsc.subcore_barrier()
            pltpu.sync_copy(acc_sh.at[pl.ds(s * RPT, RPT)],
                            out_hbm.at[p, pl.ds(s * RPT, RPT)])
            plsc.subcore_barrier()

        @pl.when(c == 0)
        def _():
            run_pass(0, sf_hbm, fs_hbm, None)
            run_pass(1, vf0_hbm, fv1_hbm, evf0_hbm)

        @pl.when(c == 1)
        def _():
            run_pass(2, vf1_hbm, fv1_hbm, evf1_hbm)
            run_pass(3, vf2_hbm, fv1_hbm, evf2_hbm)

    return sc_scatter


def _vf_split_body(vf_ref, o0_ref, o1_ref, o2_ref):
    o0_ref[...] = vf_ref[:, 0, :]
    o1_ref[...] = vf_ref[:, 1, :]
    o2_ref[...] = vf_ref[:, 2, :]


def _vf_split(vector_features, *, N, H):
    BN = 400
    grid = (N // BN,)
    out_sd = jax.ShapeDtypeStruct((N, H), jnp.float32)
    return pl.pallas_call(
        _vf_split_body,
        grid=grid,
        in_specs=[pl.BlockSpec((BN, 3, H), lambda i: (i, 0, 0))],
        out_specs=[pl.BlockSpec((BN, H), lambda i: (i, 0))] * 3,
        out_shape=[out_sd] * 3,
    )(vector_features)


def _node_body(sf_ref, vf_ref, acs_ref, ac0_ref, ac1_ref, ac2_ref,
               w3t_ref, b3_ref, w4t_ref, b4_ref, mm_ref,
               sout_ref, vout_ref, *, H):
    s = sf_ref[...] + acs_ref[0]
    v0 = vf_ref[:, 0, :] + ac0_ref[0]
    v1 = vf_ref[:, 1, :] + ac1_ref[0]
    v2 = vf_ref[:, 2, :] + ac2_ref[0]
    norms = jnp.sqrt(v0 * v0 + v1 * v1 + v2 * v2)
    comb = jnp.concatenate([s, norms], axis=-1)
    u = jnp.dot(comb, w3t_ref[...], preferred_element_type=jnp.float32) + b3_ref[...]
    u = u * jax.nn.sigmoid(u)
    upd = jnp.dot(u, w4t_ref[...], preferred_element_type=jnp.float32) + b4_ref[...]
    su = upd[:, :H]
    g1 = upd[:, H:2 * H]
    g2 = upd[:, 2 * H:]
    sout_ref[...] = s + su
    mix0 = mm_ref[0, 0] * v0 + mm_ref[0, 1] * v1 + mm_ref[0, 2] * v2
    mix1 = mm_ref[1, 0] * v0 + mm_ref[1, 1] * v1 + mm_ref[1, 2] * v2
    mix2 = mm_ref[2, 0] * v0 + mm_ref[2, 1] * v1 + mm_ref[2, 2] * v2
    vout_ref[:, 0, :] = v0 * g1 + mix0 * g2
    vout_ref[:, 1, :] = v1 * g1 + mix1 * g2
    vout_ref[:, 2, :] = v2 * g1 + mix2 * g2


def _node_update(scalar_features, vector_features, acc, W3, b3, W4, b4,
                 mixing_matrix, *, N, H):
    BN = 48 if scalar_features.shape[0] < 400 else 400
    grid = (N // BN,)
    full = lambda shape: pl.BlockSpec(shape, lambda i: (0, 0))
    acc_spec = lambda p: pl.BlockSpec((1, BN, H), lambda i, p=p: (p, i, 0))
    return pl.pallas_call(
        functools.partial(_node_body, H=H),
        grid=grid,
        in_specs=[
            pl.BlockSpec((BN, H), lambda i: (i, 0)),
            pl.BlockSpec((BN, 3, H), lambda i: (i, 0, 0)),
            acc_spec(0), acc_spec(1), acc_spec(2), acc_spec(3),
            full((2 * H, H)),
            full((1, H)),
            full((H, 3 * H)),
            full((1, 3 * H)),
            pl.BlockSpec(memory_space=pltpu.SMEM),
        ],
        out_specs=[
            pl.BlockSpec((BN, H), lambda i: (i, 0)),
            pl.BlockSpec((BN, 3, H), lambda i: (i, 0, 0)),
        ],
        out_shape=[
            jax.ShapeDtypeStruct((N, H), jnp.float32),
            jax.ShapeDtypeStruct((N, 3, H), jnp.float32),
        ],
    )(scalar_features, vector_features, acc, acc, acc, acc,
      W3.T, b3[None, :], W4.T, b4[None, :], mixing_matrix)


def kernel(scalar_features, vector_features, edge_index, edge_rbf, edge_vector,
           W1, b1, W2, b2, W3, b3, W4, b4, mixing_matrix):
    N, H = scalar_features.shape
    E, R = edge_rbf.shape
    B = 64
    chunk = NSUB * B * 4
    E_pad = ((E + chunk - 1) // chunk) * chunk
    N_pad = ((N + NSUB * 8 - 1) // (NSUB * 8)) * (NSUB * 8)

    pad = E_pad - E
    src = jnp.concatenate(
        [edge_index[0].astype(jnp.int32), jnp.zeros((pad,), jnp.int32)])
    dst = jnp.concatenate(
        [edge_index[1].astype(jnp.int32),
         jnp.full((pad,), N_pad - 1, jnp.int32)])
    idx2 = jnp.stack([src.reshape(-1, B), dst.reshape(-1, B)], axis=1)
    rbf_p = jnp.pad(edge_rbf, ((0, pad), (0, 0)))
    ev_p = jnp.pad(edge_vector, ((0, pad), (0, 0)))

    fs, fv1, evf0, evf1, evf2 = _edge_mlp(
        rbf_p, ev_p, W1, b1, W2, b2, E_pad=E_pad, H=H, R=R)

    vf0 = vector_features[:, 0, :]
    vf1 = vector_features[:, 1, :]
    vf2 = vector_features[:, 2, :]
    zeros = jnp.zeros((N_pad, H), jnp.float32)

    sc_scatter = _make_sc_scatter(E_pad, N_pad, H, B)
    acc = sc_scatter(idx2, zeros, scalar_features, vf0, vf1, vf2,
                     fs, fv1, evf0, evf1, evf2)

    return _node_update(scalar_features, vector_features, acc,
                        W3, b3, W4, b4, mixing_matrix, N=N, H=H)


# speedup vs baseline: 1.0087x; 1.0087x over previous
import functools

import jax
import jax.numpy as jnp
from jax import lax
from jax.experimental import pallas as pl
from jax.experimental.pallas import tpu as pltpu
from jax.experimental.pallas import tpu_sc as plsc

NCORE = 2
NSUB = 16
LANES = 16


def _edge_mlp_body(rbf_ref, ev_ref, w1t_ref, b1_ref, w2t_ref, b2_ref,
                   fs_ref, fv1_ref, evf0_ref, evf1_ref, evf2_ref, *, H):
    rbf = rbf_ref[...]
    h = jnp.dot(rbf, w1t_ref[...], preferred_element_type=jnp.float32) + b1_ref[...]
    h = h * jax.nn.sigmoid(h)
    fw = jnp.dot(h, w2t_ref[...], preferred_element_type=jnp.float32) + b2_ref[...]
    fs_ref[...] = fw[:, :H]
    fv1_ref[...] = fw[:, H:2 * H]
    fv2 = fw[:, 2 * H:]
    ev = ev_ref[...]
    evf0_ref[...] = ev[:, 0:1] * fv2
    evf1_ref[...] = ev[:, 1:2] * fv2
    evf2_ref[...] = ev[:, 2:3] * fv2


def _edge_mlp(rbf_p, ev_p, W1, b1, W2, b2, *, E_pad, H, R):
    BE = 512
    grid = (E_pad // BE,)
    full = lambda shape: pl.BlockSpec(shape, lambda i: (0, 0))
    out_sd = jax.ShapeDtypeStruct((E_pad, H), jnp.float32)
    return pl.pallas_call(
        functools.partial(_edge_mlp_body, H=H),
        grid=grid,
        in_specs=[
            pl.BlockSpec((BE, R), lambda i: (i, 0)),
            pl.BlockSpec((BE, 3), lambda i: (i, 0)),
            full((R, H)),
            full((1, H)),
            full((H, 3 * H)),
            full((1, 3 * H)),
        ],
        out_specs=[pl.BlockSpec((BE, H), lambda i: (i, 0))] * 5,
        out_shape=[out_sd] * 5,
    )(rbf_p, ev_p, W1.T, b1[None, :], W2.T, b2[None, :])


def _make_sc_scatter(E_pad, N_pad, H, B):
    EPT = E_pad // NSUB
    NBLK = EPT // B
    NGRP = NBLK // 4
    GBLK = E_pad // B
    RPT = N_pad // NSUB
    mesh = plsc.VectorSubcoreMesh(core_axis_name="c", subcore_axis_name="s",
                                  num_cores=NCORE, num_subcores=NSUB)

    @functools.partial(
        pl.kernel,
        out_type=jax.ShapeDtypeStruct((4, N_pad, H), jnp.float32),
        mesh=mesh,
        scratch_types=[
            pltpu.VMEM((4, 2, B), jnp.int32),
            pltpu.VMEM((2, B, H), jnp.float32),
            pltpu.VMEM((2, B, H), jnp.float32),
            pltpu.VMEM((2, B, H), jnp.float32),
            pltpu.VMEM_SHARED((N_pad, H), jnp.float32),
            pltpu.SemaphoreType.DMA, pltpu.SemaphoreType.DMA,
            pltpu.SemaphoreType.DMA, pltpu.SemaphoreType.DMA,
            pltpu.SemaphoreType.DMA, pltpu.SemaphoreType.DMA,
            pltpu.SemaphoreType.DMA, pltpu.SemaphoreType.DMA,
            pltpu.SemaphoreType.DMA, pltpu.SemaphoreType.DMA,
        ],
    )
    def sc_scatter(idx_hbm, zeros_hbm,
                   sf_hbm, vf0_hbm, vf1_hbm, vf2_hbm,
                   fs_hbm, fv1_hbm, evf0_hbm, evf1_hbm, evf2_hbm,
                   out_hbm,
                   idx_v, rows_v, mul_v, add_v, acc_sh,
                   sg0, sg1, sm0, sm1, sa0, sa1, ss0, ss1, si0, si1):
        c = lax.axis_index("c")
        s = lax.axis_index("s")
        sg = (sg0, sg1)
        sm = (sm0, sm1)
        sa = (sa0, sa1)
        ss = (ss0, ss1)
        si = (si0, si1)

        def run_pass(p, feat_hbm, mul_hbm, add_hbm):
            tb = s * EPT
            tg = s * NBLK
            pltpu.sync_copy(zeros_hbm.at[pl.ds(s * RPT, RPT)],
                            acc_sh.at[pl.ds(s * RPT, RPT)])
            plsc.subcore_barrier()

            def issue_idx(i, b4):
                pltpu.async_copy(idx_hbm.at[tg + i], idx_v.at[b4], si[b4 % 2])

            def wait_idx(b4):
                pltpu.make_async_copy(idx_hbm.at[0], idx_v.at[b4],
                                      si[b4 % 2]).wait()

            def issue_in(i, b4, bb):
                base = tb + i * B
                pltpu.async_copy(feat_hbm.at[idx_v.at[b4, 0]], rows_v.at[bb],
                                 sg[bb])
                pltpu.async_copy(mul_hbm.at[pl.ds(base, B)], mul_v.at[bb],
                                 sm[bb])
                if add_hbm is not None:
                    pltpu.async_copy(add_hbm.at[pl.ds(base, B)], add_v.at[bb],
                                     sa[bb])

            def wait_in(bb):
                pltpu.make_async_copy(feat_hbm.at[pl.ds(0, B)], rows_v.at[bb],
                                      sg[bb]).wait()
                pltpu.make_async_copy(mul_hbm.at[pl.ds(0, B)], mul_v.at[bb],
                                      sm[bb]).wait()
                if add_hbm is not None:
                    pltpu.make_async_copy(add_hbm.at[pl.ds(0, B)],
                                          add_v.at[bb], sa[bb]).wait()

            def wait_scatter(bb):
                pltpu.make_async_copy(zeros_hbm.at[pl.ds(0, B)],
                                      rows_v.at[bb], ss[bb]).wait()

            def compute(bb):
                def row_fn(r, carry):
                    for j in range(H // LANES):
                        sl = pl.ds(j * LANES, LANES)
                        m = rows_v[bb, r, sl] * mul_v[bb, r, sl]
                        if add_hbm is not None:
                            m = m + add_v[bb, r, sl]
                        rows_v[bb, r, sl] = m
                    return carry
                lax.fori_loop(0, B, row_fn, 0)

            def step(i, b4, first):
                bb = b4 % 2
                nb = (b4 + 1) % 2
                nb4 = (b4 + 1) % 4
                nn4 = (b4 + 2) % 4
                i_next = lax.rem(i + 1, NBLK)
                i_next2 = lax.rem(i + 2, NBLK)
                issue_idx(i_next2, nn4)
                if not first:
                    wait_scatter(nb)
                wait_idx(nb4)
                issue_in(i_next, nb4, nb)
                wait_in(bb)
                compute(bb)
                pltpu.async_copy(rows_v.at[bb], acc_sh.at[idx_v.at[b4, 1]],
                                 ss[bb], add=True)

            issue_idx(0, 0)
            wait_idx(0)
            issue_in(0, 0, 0)
            issue_idx(1, 1)
            for b in range(4):
                step(jnp.int32(b), b, first=(b == 0))

            def grp(g, carry):
                for b in range(4):
                    step(g * 4 + b, b, first=False)
                return carry
            lax.fori_loop(1, NGRP, grp, 0)

            wait_in(0)
            wait_idx(1)
            wait_scatter(1)

            plsc.subcore_barrier()
            pltpu.sync_copy(acc_sh.at[pl.ds(s * RPT, RPT)],
                            out_hbm.at[p, pl.ds(s * RPT, RPT)])
            plsc.subcore_barrier()

        @pl.when(c == 0)
        def _():
            run_pass(0, sf_hbm, fs_hbm, None)
            run_pass(1, vf0_hbm, fv1_hbm, evf0_hbm)

        @pl.when(c == 1)
        def _():
            run_pass(2, vf1_hbm, fv1_hbm, evf1_hbm)
            run_pass(3, vf2_hbm, fv1_hbm, evf2_hbm)

    return sc_scatter


def _vf_split_body(vf_ref, o0_ref, o1_ref, o2_ref):
    o0_ref[...] = vf_ref[:, 0, :]
    o1_ref[...] = vf_ref[:, 1, :]
    o2_ref[...] = vf_ref[:, 2, :]


def _vf_split(vector_features, *, N, H):
    BN = 400
    grid = (N // BN,)
    out_sd = jax.ShapeDtypeStruct((N, H), jnp.float32)
    return pl.pallas_call(
        _vf_split_body,
        grid=grid,
        in_specs=[pl.BlockSpec((BN, 3, H), lambda i: (i, 0, 0))],
        out_specs=[pl.BlockSpec((BN, H), lambda i: (i, 0))] * 3,
        out_shape=[out_sd] * 3,
    )(vector_features)


def _node_body(sf_ref, vf_ref, acs_ref, ac0_ref, ac1_ref, ac2_ref,
               w3t_ref, b3_ref, w4t_ref, b4_ref, mm_ref,
               sout_ref, vout_ref, *, H):
    s = sf_ref[...] + acs_ref[0]
    v0 = vf_ref[:, 0, :] + ac0_ref[0]
    v1 = vf_ref[:, 1, :] + ac1_ref[0]
    v2 = vf_ref[:, 2, :] + ac2_ref[0]
    norms = jnp.sqrt(v0 * v0 + v1 * v1 + v2 * v2)
    comb = jnp.concatenate([s, norms], axis=-1)
    u = jnp.dot(comb, w3t_ref[...], preferred_element_type=jnp.float32) + b3_ref[...]
    u = u * jax.nn.sigmoid(u)
    upd = jnp.dot(u, w4t_ref[...], preferred_element_type=jnp.float32) + b4_ref[...]
    su = upd[:, :H]
    g1 = upd[:, H:2 * H]
    g2 = upd[:, 2 * H:]
    sout_ref[...] = s + su
    mix0 = mm_ref[0, 0] * v0 + mm_ref[0, 1] * v1 + mm_ref[0, 2] * v2
    mix1 = mm_ref[1, 0] * v0 + mm_ref[1, 1] * v1 + mm_ref[1, 2] * v2
    mix2 = mm_ref[2, 0] * v0 + mm_ref[2, 1] * v1 + mm_ref[2, 2] * v2
    vout_ref[:, 0, :] = v0 * g1 + mix0 * g2
    vout_ref[:, 1, :] = v1 * g1 + mix1 * g2
    vout_ref[:, 2, :] = v2 * g1 + mix2 * g2


def _node_update(scalar_features, vector_features, acc, W3, b3, W4, b4,
                 mixing_matrix, *, N, H):
    BN = 48 if scalar_features.shape[0] < 400 else 400
    grid = (N // BN,)
    full = lambda shape: pl.BlockSpec(shape, lambda i: (0, 0))
    acc_spec = lambda p: pl.BlockSpec((1, BN, H), lambda i, p=p: (p, i, 0))
    return pl.pallas_call(
        functools.partial(_node_body, H=H),
        grid=grid,
        in_specs=[
            pl.BlockSpec((BN, H), lambda i: (i, 0)),
            pl.BlockSpec((BN, 3, H), lambda i: (i, 0, 0)),
            acc_spec(0), acc_spec(1), acc_spec(2), acc_spec(3),
            full((2 * H, H)),
            full((1, H)),
            full((H, 3 * H)),
            full((1, 3 * H)),
            pl.BlockSpec(memory_space=pltpu.SMEM),
        ],
        out_specs=[
            pl.BlockSpec((BN, H), lambda i: (i, 0)),
            pl.BlockSpec((BN, 3, H), lambda i: (i, 0, 0)),
        ],
        out_shape=[
            jax.ShapeDtypeStruct((N, H), jnp.float32),
            jax.ShapeDtypeStruct((N, 3, H), jnp.float32),
        ],
    )(scalar_features, vector_features, acc, acc, acc, acc,
      W3.T, b3[None, :], W4.T, b4[None, :], mixing_matrix)


def kernel(scalar_features, vector_features, edge_index, edge_rbf, edge_vector,
           W1, b1, W2, b2, W3, b3, W4, b4, mixing_matrix):
    N, H = scalar_features.shape
    E, R = edge_rbf.shape
    B = 64
    chunk = NSUB * B * 4
    E_pad = ((E + chunk - 1) // chunk) * chunk
    N_pad = ((N + NSUB * 8 - 1) // (NSUB * 8)) * (NSUB * 8)

    pad = E_pad - E
    src = jnp.concatenate(
        [edge_index[0].astype(jnp.int32), jnp.zeros((pad,), jnp.int32)])
    dst = jnp.concatenate(
        [edge_index[1].astype(jnp.int32),
         jnp.full((pad,), N_pad - 1, jnp.int32)])
    idx2 = jnp.stack([src.reshape(-1, B), dst.reshape(-1, B)], axis=1)
    rbf_p = jnp.pad(edge_rbf, ((0, pad), (0, 0)))
    ev_p = jnp.pad(edge_vector, ((0, pad), (0, 0)))

    fs, fv1, evf0, evf1, evf2 = _edge_mlp(
        rbf_p, ev_p, W1, b1, W2, b2, E_pad=E_pad, H=H, R=R)

    vf0 = vector_features[:, 0, :]
    vf1 = vector_features[:, 1, :]
    vf2 = vector_features[:, 2, :]
    zeros = jnp.zeros((N_pad, H), jnp.float32)

    sc_scatter = _make_sc_scatter(E_pad, N_pad, H, B)
    acc = sc_scatter(idx2, zeros, scalar_features, vf0, vf1, vf2,
                     fs, fv1, evf0, evf1, evf2)

    return _node_update(scalar_features, vector_features, acc,
                        W3, b3, W4, b4, mixing_matrix, N=N, H=H)
